# baseline (speedup 1.0000x reference)
import functools

import jax
import jax.numpy as jnp
from jax.experimental import pallas as pl
from jax.experimental.pallas import tpu as pltpu


def _moe_dense_body(x_ref, wg_ref, w1_ref, w2_ref, out_ref, logits_ref,
                    acc_ref, comb_ref, *, n_e, n_f):
    e = pl.program_id(1)
    f = pl.program_id(2)
    tm = x_ref.shape[0]
    n_exp = wg_ref.shape[0]

    @pl.when((e == 0) & (f == 0))
    def _router():
        xt = x_ref[...]
        logits = jax.lax.dot_general(
            xt, wg_ref[...], (((1,), (1,)), ((), ())),
            preferred_element_type=jnp.float32)
        logits_ref[...] = logits
        ids = jax.lax.broadcasted_iota(jnp.int32, (tm, n_exp), 1)
        m1 = jnp.max(logits, axis=1, keepdims=True)
        i1 = jnp.min(jnp.where(logits == m1, ids, n_exp), axis=1, keepdims=True)
        masked = jnp.where(ids == i1, -jnp.inf, logits)
        m2 = jnp.max(masked, axis=1, keepdims=True)
        i2 = jnp.min(jnp.where(masked == m2, ids, n_exp), axis=1, keepdims=True)
        z = jnp.exp(m2 - m1)
        w_hi = 1.0 / (1.0 + z)
        w_lo = z / (1.0 + z)
        comb_ref[...] = (jnp.where(ids == i1, w_hi, 0.0)
                         + jnp.where(ids == i2, w_lo, 0.0))
        acc_ref[...] = jnp.zeros_like(acc_ref)

    xt = x_ref[...].astype(jnp.bfloat16)
    h = jax.lax.dot_general(
        xt, w1_ref[0], (((1,), (1,)), ((), ())),
        preferred_element_type=jnp.float32)
    h = jnp.maximum(h, 0.0)
    ids = jax.lax.broadcasted_iota(jnp.int32, (tm, n_exp), 1)
    c = jnp.sum(jnp.where(ids == e, comb_ref[...], 0.0), axis=1,
                keepdims=True)
    hw = (h * c).astype(jnp.bfloat16)
    acc_ref[...] += jax.lax.dot_general(
        hw, w2_ref[0], (((1,), (1,)), ((), ())),
        preferred_element_type=jnp.float32)

    @pl.when((e == n_e - 1) & (f == n_f - 1))
    def _emit():
        out_ref[...] = acc_ref[...]


def kernel(x, Wg, W1, W2):
    b, t, hdim = x.shape
    n_exp, ff, _ = W1.shape
    bt = b * t
    tm = min(512, bt)
    fc = min(1024, ff)
    n_t, n_f = bt // tm, ff // fc

    xf = x.reshape(bt, hdim)
    W1 = W1.astype(jnp.bfloat16)
    W2 = W2.astype(jnp.bfloat16)
    body = functools.partial(_moe_dense_body, n_e=n_exp, n_f=n_f)
    out, logits = pl.pallas_call(
        body,
        grid=(n_t, n_exp, n_f),
        in_specs=[
            pl.BlockSpec((tm, hdim), lambda ti, e, f: (ti, 0)),
            pl.BlockSpec((n_exp, hdim), lambda ti, e, f: (0, 0)),
            pl.BlockSpec((1, fc, hdim), lambda ti, e, f: (e, f, 0)),
            pl.BlockSpec((1, hdim, fc), lambda ti, e, f: (e, 0, f)),
        ],
        out_specs=[
            pl.BlockSpec((tm, hdim), lambda ti, e, f: (ti, 0)),
            pl.BlockSpec((tm, n_exp), lambda ti, e, f: (ti, 0)),
        ],
        out_shape=[
            jax.ShapeDtypeStruct((bt, hdim), jnp.float32),
            jax.ShapeDtypeStruct((bt, n_exp), jnp.float32),
        ],
        scratch_shapes=[
            pltpu.VMEM((tm, hdim), jnp.float32),
            pltpu.VMEM((tm, n_exp), jnp.float32),
        ],
        compiler_params=pltpu.CompilerParams(
            dimension_semantics=("parallel", "arbitrary", "arbitrary")),
    )(xf, Wg, W1, W2)
    return out.reshape(b, t, hdim), logits.reshape(b, t, n_exp)


# speedup vs baseline: 1.7564x; 1.7564x over previous
import functools

import jax
import jax.numpy as jnp
from jax import lax
from jax.experimental import pallas as pl
from jax.experimental.pallas import tpu as pltpu
from jax.experimental.pallas import tpu_sc as plsc

TM = 512
FC = 2048
CH = 512
EP = 16


def _router_body(x_ref, wg_ref, logits_ref, e0_ref, e1_ref, r0_ref, r1_ref,
                 w0_ref, w1_ref, start_ref, eid_ref, valid_ref, carry_ref,
                 *, n_c, n_e, g_tiles):
    c = pl.program_id(0)
    ch = x_ref.shape[0]

    logits = lax.dot_general(x_ref[...], wg_ref[...], (((1,), (1,)), ((), ())),
                             preferred_element_type=jnp.float32)
    logits_ref[...] = logits
    ids = lax.broadcasted_iota(jnp.int32, (ch, n_e), 1)
    m1 = jnp.max(logits, axis=1, keepdims=True)
    i1 = jnp.min(jnp.where(logits == m1, ids, n_e), axis=1, keepdims=True)
    masked = jnp.where(ids == i1, -jnp.inf, logits)
    m2 = jnp.max(masked, axis=1, keepdims=True)
    i2 = jnp.min(jnp.where(masked == m2, ids, n_e), axis=1, keepdims=True)
    z = jnp.exp(m2 - m1)
    w_hi = 1.0 / (1.0 + z)
    w_lo = z / (1.0 + z)

    @pl.when(c == 0)
    def _init():
        carry_ref[...] = jnp.zeros_like(carry_ref)

    ids16 = lax.broadcasted_iota(jnp.int32, (ch, EP), 1)
    oh0 = (ids16 == i1).astype(jnp.float32)
    oh1 = (ids16 == i2).astype(jnp.float32)
    ohb = oh0 + oh1
    ri = lax.broadcasted_iota(jnp.int32, (ch, ch), 0)
    ci = lax.broadcasted_iota(jnp.int32, (ch, ch), 1)
    tril = (ri > ci).astype(jnp.float32)
    cex = lax.dot_general(tril, ohb, (((1,), (0,)), ((), ())),
                          preferred_element_type=jnp.float32)
    base = cex + carry_ref[...]
    rank0 = jnp.sum(oh0 * base, axis=1, keepdims=True)
    rank1 = jnp.sum(oh1 * base, axis=1, keepdims=True)

    e0_ref[0] = i1
    e1_ref[0] = i2
    r0_ref[0] = rank0.astype(jnp.int32)
    r1_ref[0] = rank1.astype(jnp.int32)
    w0_ref[0] = w_hi
    w1_ref[0] = w_lo
    carry_ref[...] += jnp.sum(ohb, axis=0, keepdims=True)

    @pl.when(c == n_c - 1)
    def _finalize():
        cnt = carry_ref[...]
        ntiles = jnp.floor((cnt + (TM - 1)) / TM)
        er = lax.broadcasted_iota(jnp.int32, (EP, EP), 0)
        ec = lax.broadcasted_iota(jnp.int32, (EP, EP), 1)
        mexcl = (er < ec).astype(jnp.float32)
        st = lax.dot_general(ntiles, mexcl, (((1,), (0,)), ((), ())),
                             preferred_element_type=jnp.float32)
        start_ref[0] = (st * TM).astype(jnp.int32)
        seg_end = st + ntiles
        lane = lax.broadcasted_iota(jnp.int32, (1, EP), 1)
        gi = lax.broadcasted_iota(jnp.int32, (1, g_tiles), 1).astype(
            jnp.float32)
        te = jnp.zeros((1, g_tiles), jnp.float32)
        for e in range(n_e):
            se = jnp.sum(jnp.where(lane == e, seg_end, 0.0))
            te = te + (gi >= se).astype(jnp.float32)
        eid_ref[0] = jnp.minimum(te, float(n_e - 1)).astype(jnp.int32)
        total = jnp.max(seg_end)
        valid_ref[0] = (gi < total).astype(jnp.int32)


def _router_call(xf, Wg, g_tiles):
    bt, hdim = xf.shape
    n_e = Wg.shape[0]
    n_c = bt // CH
    body = functools.partial(_router_body, n_c=n_c, n_e=n_e, g_tiles=g_tiles)
    out_shape = [
        jax.ShapeDtypeStruct((bt, n_e), jnp.float32),
        jax.ShapeDtypeStruct((n_c, CH, 1), jnp.int32),
        jax.ShapeDtypeStruct((n_c, CH, 1), jnp.int32),
        jax.ShapeDtypeStruct((n_c, CH, 1), jnp.int32),
        jax.ShapeDtypeStruct((n_c, CH, 1), jnp.int32),
        jax.ShapeDtypeStruct((n_c, CH, 1), jnp.float32),
        jax.ShapeDtypeStruct((n_c, CH, 1), jnp.float32),
        jax.ShapeDtypeStruct((1, 1, EP), jnp.int32),
        jax.ShapeDtypeStruct((1, 1, g_tiles), jnp.int32),
        jax.ShapeDtypeStruct((1, 1, g_tiles), jnp.int32),
    ]
    seq = lambda c: (c, 0, 0)
    z3 = lambda c: (0, 0, 0)
    return pl.pallas_call(
        body,
        grid=(n_c,),
        in_specs=[
            pl.BlockSpec((CH, hdim), lambda c: (c, 0)),
            pl.BlockSpec((n_e, hdim), lambda c: (0, 0)),
        ],
        out_specs=[
            pl.BlockSpec((CH, n_e), lambda c: (c, 0)),
            pl.BlockSpec((1, CH, 1), seq),
            pl.BlockSpec((1, CH, 1), seq),
            pl.BlockSpec((1, CH, 1), seq),
            pl.BlockSpec((1, CH, 1), seq),
            pl.BlockSpec((1, CH, 1), seq),
            pl.BlockSpec((1, CH, 1), seq),
            pl.BlockSpec((1, 1, EP), z3),
            pl.BlockSpec((1, 1, g_tiles), z3),
            pl.BlockSpec((1, 1, g_tiles), z3),
        ],
        out_shape=out_shape,
        scratch_shapes=[pltpu.VMEM((1, EP), jnp.float32)],
        compiler_params=pltpu.CompilerParams(
            dimension_semantics=("arbitrary",)),
    )(xf, Wg)


def _dispatch_body(x_hbm, e0_hbm, e1_hbm, r0_hbm, r1_hbm, w0_hbm, w1_hbm,
                   start_hbm, xs_hbm, sw_hbm, xbuf, i0buf, i1buf, ebuf, rbuf,
                   wbuf, startv, sem, *, n_tok, n_sub):
    ct = xbuf.shape[0]
    wid = lax.axis_index("s") * 2 + lax.axis_index("c")
    per_w = n_tok // n_sub
    n_chunk = per_w // ct
    pltpu.sync_copy(start_hbm, startv)

    def chunk(j, carry):
        base = wid * per_w + j * ct
        pltpu.sync_copy(x_hbm.at[pl.ds(base, ct)], xbuf)
        pltpu.sync_copy(e0_hbm.at[pl.ds(base, ct)], ebuf)
        pltpu.sync_copy(r0_hbm.at[pl.ds(base, ct)], rbuf)
        for v in range(ct // 16):
            sl = pl.ds(v * 16, 16)
            sv = plsc.load_gather(startv, [ebuf[sl]])
            i0buf[sl] = sv + rbuf[sl]
        pltpu.sync_copy(e1_hbm.at[pl.ds(base, ct)], ebuf)
        pltpu.sync_copy(r1_hbm.at[pl.ds(base, ct)], rbuf)
        for v in range(ct // 16):
            sl = pl.ds(v * 16, 16)
            sv = plsc.load_gather(startv, [ebuf[sl]])
            i1buf[sl] = sv + rbuf[sl]
        c0 = pltpu.async_copy(xbuf, xs_hbm.at[i0buf], sem)
        c1 = pltpu.async_copy(xbuf, xs_hbm.at[i1buf], sem)
        pltpu.sync_copy(w0_hbm.at[pl.ds(base, ct)], wbuf)
        c2 = pltpu.async_copy(wbuf, sw_hbm.at[i0buf], sem)
        c0.wait()
        c1.wait()
        c2.wait()
        pltpu.sync_copy(w1_hbm.at[pl.ds(base, ct)], wbuf)
        pltpu.async_copy(wbuf, sw_hbm.at[i1buf], sem).wait()
        return carry

    lax.fori_loop(0, n_chunk, chunk, 0)


def _dispatch_call(xf, e0, e1, r0, r1, w0, w1, start, n_slots):
    n_tok, hdim = xf.shape
    ct = 32
    mesh = plsc.VectorSubcoreMesh(core_axis_name="c", subcore_axis_name="s")
    body = functools.partial(_dispatch_body, n_tok=n_tok, n_sub=32)
    f = pl.kernel(
        body,
        out_type=[
            jax.ShapeDtypeStruct((n_slots, hdim), jnp.float32),
            jax.ShapeDtypeStruct((n_slots,), jnp.float32),
        ],
        mesh=mesh,
        scratch_types=[
            pltpu.VMEM((ct, hdim), jnp.float32),
            pltpu.VMEM((ct,), jnp.int32),
            pltpu.VMEM((ct,), jnp.int32),
            pltpu.VMEM((ct,), jnp.int32),
            pltpu.VMEM((ct,), jnp.int32),
            pltpu.VMEM((ct,), jnp.float32),
            pltpu.VMEM((EP,), jnp.int32),
            pltpu.SemaphoreType.DMA,
        ],
        compiler_params=pltpu.CompilerParams(needs_layout_passes=False),
    )
    return f(xf, e0, e1, r0, r1, w0, w1, start)


def _ffn_body(eid_ref, valid_ref, xs_ref, sw_ref, w1_ref, w2_ref, y_ref,
              h_ref):
    g = pl.program_id(1)

    @pl.when(valid_ref[g] == 1)
    def _compute():
        h = lax.dot_general(xs_ref[...], w1_ref[0],
                            (((1,), (1,)), ((), ())),
                            preferred_element_type=jnp.float32)
        h_ref[...] = jnp.maximum(h, 0.0)
        y = lax.dot_general(h_ref[...], w2_ref[0],
                            (((1,), (1,)), ((), ())),
                            preferred_element_type=jnp.float32)
        y_ref[0] = y * sw_ref[0]


def _ffn_call(xs, sw, W1, W2, g_tiles, eid, valid):
    n_slots, hdim = xs.shape
    n_e, ff, _ = W1.shape
    n_f = ff // FC
    sw3 = sw.reshape(g_tiles, TM, 1)
    grid_spec = pltpu.PrefetchScalarGridSpec(
        num_scalar_prefetch=2,
        grid=(n_f, g_tiles),
        in_specs=[
            pl.BlockSpec((TM, hdim), lambda f, g, eid, val: (g, 0)),
            pl.BlockSpec((1, TM, 1), lambda f, g, eid, val: (g, 0, 0)),
            pl.BlockSpec((1, FC, hdim), lambda f, g, eid, val: (eid[g], f, 0)),
            pl.BlockSpec((1, hdim, FC), lambda f, g, eid, val: (eid[g], 0, f)),
        ],
        out_specs=pl.BlockSpec((1, TM, hdim),
                               lambda f, g, eid, val: (f, g, 0)),
        scratch_shapes=[pltpu.VMEM((TM, FC), jnp.float32)],
    )
    return pl.pallas_call(
        _ffn_body,
        grid_spec=grid_spec,
        out_shape=jax.ShapeDtypeStruct((n_f, n_slots, hdim), jnp.float32),
        compiler_params=pltpu.CompilerParams(
            dimension_semantics=("arbitrary", "arbitrary")),
    )(eid, valid, xs, sw3, W1, W2)


def _combine_body(y_hbm, e0_hbm, e1_hbm, r0_hbm, r1_hbm, start_hbm, z_hbm,
                  b0, b1, b2, b3, i0buf, i1buf, i2buf, i3buf, ebuf, rbuf,
                  startv, sem, *, n_tok, n_sub, n_slots):
    ct = b0.shape[0]
    wid = lax.axis_index("s") * 2 + lax.axis_index("c")
    per_w = n_tok // n_sub
    n_chunk = per_w // ct
    pltpu.sync_copy(start_hbm, startv)

    def chunk(j, carry):
        base = wid * per_w + j * ct
        pltpu.sync_copy(e0_hbm.at[pl.ds(base, ct)], ebuf)
        pltpu.sync_copy(r0_hbm.at[pl.ds(base, ct)], rbuf)
        sv = plsc.load_gather(startv, [ebuf[...]])
        i0buf[...] = sv + rbuf[...]
        i2buf[...] = i0buf[...] + n_slots
        pltpu.sync_copy(e1_hbm.at[pl.ds(base, ct)], ebuf)
        pltpu.sync_copy(r1_hbm.at[pl.ds(base, ct)], rbuf)
        sv1 = plsc.load_gather(startv, [ebuf[...]])
        i1buf[...] = sv1 + rbuf[...]
        i3buf[...] = i1buf[...] + n_slots
        c0 = pltpu.async_copy(y_hbm.at[i0buf], b0, sem)
        c1 = pltpu.async_copy(y_hbm.at[i1buf], b1, sem)
        c2 = pltpu.async_copy(y_hbm.at[i2buf], b2, sem)
        c3 = pltpu.async_copy(y_hbm.at[i3buf], b3, sem)
        c0.wait()
        c1.wait()
        c2.wait()
        c3.wait()
        pltpu.sync_copy(b0, z_hbm.at[0, pl.ds(base, ct)])
        pltpu.sync_copy(b1, z_hbm.at[1, pl.ds(base, ct)])
        pltpu.sync_copy(b2, z_hbm.at[2, pl.ds(base, ct)])
        pltpu.sync_copy(b3, z_hbm.at[3, pl.ds(base, ct)])
        return carry

    lax.fori_loop(0, n_chunk, chunk, 0)


def _combine_call(y2, e0, e1, r0, r1, start, n_tok, hdim, n_slots):
    ct = 16
    mesh = plsc.VectorSubcoreMesh(core_axis_name="c", subcore_axis_name="s")
    body = functools.partial(_combine_body, n_tok=n_tok, n_sub=32,
                             n_slots=n_slots)
    f = pl.kernel(
        body,
        out_type=jax.ShapeDtypeStruct((4, n_tok, hdim), jnp.float32),
        mesh=mesh,
        scratch_types=[
            pltpu.VMEM((ct, hdim), jnp.float32),
            pltpu.VMEM((ct, hdim), jnp.float32),
            pltpu.VMEM((ct, hdim), jnp.float32),
            pltpu.VMEM((ct, hdim), jnp.float32),
            pltpu.VMEM((ct,), jnp.int32),
            pltpu.VMEM((ct,), jnp.int32),
            pltpu.VMEM((ct,), jnp.int32),
            pltpu.VMEM((ct,), jnp.int32),
            pltpu.VMEM((ct,), jnp.int32),
            pltpu.VMEM((ct,), jnp.int32),
            pltpu.VMEM((EP,), jnp.int32),
            pltpu.SemaphoreType.DMA,
        ],
        compiler_params=pltpu.CompilerParams(needs_layout_passes=False),
    )
    z = f(y2, e0, e1, r0, r1, start)
    return _sum4_call(z)


def _sum4_body(z_ref, out_ref):
    out_ref[...] = (z_ref[0] + z_ref[1]) + (z_ref[2] + z_ref[3])


def _sum4_call(z):
    _, n_tok, hdim = z.shape
    tmo = 512
    return pl.pallas_call(
        _sum4_body,
        grid=(n_tok // tmo,),
        in_specs=[pl.BlockSpec((4, tmo, hdim), lambda t: (0, t, 0))],
        out_specs=pl.BlockSpec((tmo, hdim), lambda t: (t, 0)),
        out_shape=jax.ShapeDtypeStruct((n_tok, hdim), jnp.float32),
        compiler_params=pltpu.CompilerParams(
            dimension_semantics=("parallel",)),
    )(z)


def kernel(x, Wg, W1, W2):
    b, t, hdim = x.shape
    n_e, ff, _ = W1.shape
    bt = b * t
    n_assign = 2 * bt
    g_tiles = n_assign // TM + n_e
    n_slots = g_tiles * TM

    xf = x.reshape(bt, hdim)
    (logits, e0, e1, r0, r1, w0, w1, start, eid, valid) = _router_call(
        xf, Wg, g_tiles)
    e0 = e0.reshape(bt)
    e1 = e1.reshape(bt)
    r0 = r0.reshape(bt)
    r1 = r1.reshape(bt)
    w0 = w0.reshape(bt)
    w1 = w1.reshape(bt)
    start1 = start.reshape(EP)
    eid1 = eid.reshape(g_tiles)
    valid1 = valid.reshape(g_tiles)

    xs, sw = _dispatch_call(xf, e0, e1, r0, r1, w0, w1, start1, n_slots)
    y = _ffn_call(xs, sw, W1, W2, g_tiles, eid1, valid1)
    y2 = y.reshape(2 * n_slots, hdim)
    out = _combine_call(y2, e0, e1, r0, r1, start1, bt, hdim, n_slots)
    return out.reshape(b, t, hdim), logits.reshape(b, t, n_e)
